# Initial kernel scaffold; baseline (speedup 1.0000x reference)
import jax
import jax.numpy as jnp
from jax.experimental import pallas as pl


def kernel(x, weight):
    raise NotImplementedError("write your pallas kernel here")


# speedup vs baseline: 1.0946x; 1.0946x over previous
import jax
import jax.numpy as jnp
from jax import lax
from jax.experimental import pallas as pl
from jax.experimental.pallas import tpu as pltpu
from jax.experimental.pallas import tpu_sc as plsc

NUM_EMBEDDINGS = 1000000
EMBEDDING_DIM = 32

_info = plsc.get_sparse_core_info()
_NC, _NS = _info.num_cores, _info.num_subcores
_NW = _NC * _NS

_B = 16384 * 50
_PER_W = _B // _NW
_K = 8
_CHUNK = _K * 128
_G = _PER_W // _CHUNK


def _body(idx_hbm, table_hbm, out_hbm, idx_v, rows_v, gsem):
    wid = lax.axis_index("s") * _NC + lax.axis_index("c")
    base = wid * _PER_W

    def group(g, carry):
        row0 = base + g * _CHUNK
        pltpu.sync_copy(idx_hbm.at[pl.ds(pl.multiple_of(row0 // 128, 8), _K)], idx_v)
        copies = [
            pltpu.async_copy(
                table_hbm.at[idx_v.at[j]],
                rows_v.at[pl.ds(j * 128, 128)],
                gsem,
            )
            for j in range(_K)
        ]
        for c in copies:
            c.wait()
        pltpu.sync_copy(rows_v, out_hbm.at[pl.ds(row0, _CHUNK)])
        return carry

    lax.fori_loop(0, _G, group, 0)


def kernel(x, weight):
    idx = x.reshape(_B // 128, 128).astype(jnp.int32)
    launch = pl.kernel(
        _body,
        out_type=jax.ShapeDtypeStruct((_B, EMBEDDING_DIM), jnp.float32),
        mesh=plsc.VectorSubcoreMesh(core_axis_name="c", subcore_axis_name="s"),
        compiler_params=pltpu.CompilerParams(use_tc_tiling_on_sc=False),
        scratch_types=[
            pltpu.VMEM((_K, 128), jnp.int32),
            pltpu.VMEM((_CHUNK, EMBEDDING_DIM), jnp.float32),
            pltpu.SemaphoreType.DMA,
        ],
    )
    out = launch(idx, weight)
    return out.reshape(16384, 50, EMBEDDING_DIM)


# speedup vs baseline: 1.1136x; 1.0174x over previous
import jax
import jax.numpy as jnp
from jax import lax
from jax.experimental import pallas as pl
from jax.experimental.pallas import tpu as pltpu
from jax.experimental.pallas import tpu_sc as plsc

NUM_EMBEDDINGS = 1000000
EMBEDDING_DIM = 32

_info = plsc.get_sparse_core_info()
_NC, _NS = _info.num_cores, _info.num_subcores
_NW = _NC * _NS

_B = 16384 * 50
_PER_W = _B // _NW
_K = 5
_C = _K * 128
_G = _PER_W // _C
_NBUF = 4
_P = _G // _NBUF


def _mo(v, m):
    return v if isinstance(v, int) else pl.multiple_of(v, m)


def _body(idx_hbm, table_hbm, out_hbm, idx_v, rows, *sems):
    gsem = sems[:_NBUF]
    wsem = sems[_NBUF:]
    wid = lax.axis_index("s") * _NC + lax.axis_index("c")
    base = _mo(wid * _PER_W, 8)
    pltpu.sync_copy(idx_hbm.at[pl.ds(base, _PER_W)], idx_v)

    def fire(g, b):
        for j in range(_K):
            off = _mo(g * _C + j * 128, 8)
            pltpu.async_copy(
                table_hbm.at[idx_v.at[pl.ds(off, 128)]],
                rows.at[b, pl.ds(j * 128, 128)],
                gsem[b],
            )

    def drain_gather(b):
        for j in range(_K):
            pltpu.make_async_copy(
                table_hbm.at[pl.ds(0, 128)],
                rows.at[b, pl.ds(j * 128, 128)],
                gsem[b],
            ).wait()

    def write(g, b):
        row0 = _mo(base + g * _C, 8)
        pltpu.async_copy(rows.at[b], out_hbm.at[pl.ds(row0, _C)], wsem[b])

    def drain_write(b):
        pltpu.make_async_copy(rows.at[b], out_hbm.at[pl.ds(base, _C)], wsem[b]).wait()

    for b in range(_NBUF):
        fire(b, b)

    def turn(p, carry):
        for b in range(_NBUF):
            g = p * _NBUF + b
            drain_gather(b)
            write(g, b)
            drain_write(b)
            fire(g + _NBUF, b)
        return carry

    lax.fori_loop(0, _P - 1, turn, 0)

    for b in range(_NBUF):
        g = (_P - 1) * _NBUF + b
        drain_gather(b)
        write(g, b)
    for b in range(_NBUF):
        drain_write(b)


def kernel(x, weight):
    idx = x.reshape(_B).astype(jnp.int32)
    launch = pl.kernel(
        _body,
        out_type=jax.ShapeDtypeStruct((_B, EMBEDDING_DIM), jnp.float32),
        mesh=plsc.VectorSubcoreMesh(core_axis_name="c", subcore_axis_name="s"),
        compiler_params=pltpu.CompilerParams(use_tc_tiling_on_sc=False),
        scratch_types=[
            pltpu.VMEM((_PER_W,), jnp.int32),
            pltpu.VMEM((_NBUF, _C, EMBEDDING_DIM), jnp.float32),
        ] + [pltpu.SemaphoreType.DMA] * (2 * _NBUF),
    )
    out = launch(idx, weight)
    return out.reshape(16384, 50, EMBEDDING_DIM)


# speedup vs baseline: 1.1142x; 1.0006x over previous
import jax
import jax.numpy as jnp
from jax import lax
from jax.experimental import pallas as pl
from jax.experimental.pallas import tpu as pltpu
from jax.experimental.pallas import tpu_sc as plsc

NUM_EMBEDDINGS = 1000000
EMBEDDING_DIM = 32

_info = plsc.get_sparse_core_info()
_NC, _NS = _info.num_cores, _info.num_subcores
_NW = _NC * _NS

_B = 16384 * 50
_PER_W = _B // _NW
_K = 1
_C = 640
_L = _C // _K
_G = _PER_W // _C
_NBUF = 4
_P = _G // _NBUF


def _mo(v, m):
    return v if isinstance(v, int) else pl.multiple_of(v, m)


def _body(idx_hbm, table_hbm, out_hbm, idx_v, rows, *sems):
    gsem = sems[:_NBUF]
    wsem = sems[_NBUF:]
    wid = lax.axis_index("s") * _NC + lax.axis_index("c")
    base = _mo(wid * _PER_W, 8)
    pltpu.sync_copy(idx_hbm.at[pl.ds(base, _PER_W)], idx_v)

    def fire(g, b):
        for j in range(_K):
            off = _mo(g * _C + j * _L, 8)
            pltpu.async_copy(
                table_hbm.at[idx_v.at[pl.ds(off, _L)]],
                rows.at[b, pl.ds(j * _L, _L)],
                gsem[b],
            )

    def drain_gather(b):
        for j in range(_K):
            pltpu.make_async_copy(
                table_hbm.at[pl.ds(0, _L)],
                rows.at[b, pl.ds(j * _L, _L)],
                gsem[b],
            ).wait()

    def write(g, b):
        row0 = _mo(base + g * _C, 8)
        pltpu.async_copy(rows.at[b], out_hbm.at[pl.ds(row0, _C)], wsem[b])

    def drain_write(b):
        pltpu.make_async_copy(rows.at[b], out_hbm.at[pl.ds(base, _C)], wsem[b]).wait()

    for b in range(_NBUF):
        fire(b, b)

    def turn(p, carry):
        for b in range(_NBUF):
            g = p * _NBUF + b
            drain_gather(b)
            write(g, b)
            drain_write(b)
            fire(g + _NBUF, b)
        return carry

    lax.fori_loop(0, _P - 1, turn, 0)

    for b in range(_NBUF):
        g = (_P - 1) * _NBUF + b
        drain_gather(b)
        write(g, b)
    for b in range(_NBUF):
        drain_write(b)


def kernel(x, weight):
    idx = x.reshape(_B).astype(jnp.int32)
    launch = pl.kernel(
        _body,
        out_type=jax.ShapeDtypeStruct((_B, EMBEDDING_DIM), jnp.float32),
        mesh=plsc.VectorSubcoreMesh(core_axis_name="c", subcore_axis_name="s"),
        compiler_params=pltpu.CompilerParams(use_tc_tiling_on_sc=False),
        scratch_types=[
            pltpu.VMEM((_PER_W,), jnp.int32),
            pltpu.VMEM((_NBUF, _C, EMBEDDING_DIM), jnp.float32),
        ] + [pltpu.SemaphoreType.DMA] * (2 * _NBUF),
    )
    out = launch(idx, weight)
    return out.reshape(16384, 50, EMBEDDING_DIM)
